# Initial kernel scaffold; baseline (speedup 1.0000x reference)
import jax
import jax.numpy as jnp
from jax.experimental import pallas as pl


def kernel(z, pos, batch, atomref_table, emb_feat, W1, b1, W2, b2):
    raise NotImplementedError("write your pallas kernel here")


# speedup vs baseline: 13.9764x; 13.9764x over previous
import jax
import jax.numpy as jnp
from jax import lax
from jax.experimental import pallas as pl

N_ATOMS = 100000
MAX_Z = 100
D_HID = 256
ZPAD = 128
BLK = 2048
N_PAD = 100352
GRID = N_PAD // BLK


def _tc_body(z_ref, posT_ref, tableT_ref, w1T_ref, w2T_ref, aref_ref, out_ref):
    z = z_ref[0, 0, :]
    species = lax.broadcasted_iota(jnp.int32, (ZPAD, BLK), 0)
    onehotT = (species == z[None, :]).astype(jnp.float32)
    gT = jnp.dot(tableT_ref[...], onehotT, preferred_element_type=jnp.float32)
    pT = (w1T_ref[:, 0:1] * posT_ref[0:1, :]
          + w1T_ref[:, 1:2] * posT_ref[1:2, :]
          + w1T_ref[:, 2:3] * posT_ref[2:3, :])
    hT = jnp.tanh(pT + gT)
    xT = jnp.dot(w2T_ref[...], hT, preferred_element_type=jnp.float32)
    arefT = jnp.dot(aref_ref[...], onehotT, preferred_element_type=jnp.float32)
    out_ref[0, :, :] = xT + arefT


def kernel(z, pos, batch, atomref_table, emb_feat, W1, b1, W2, b2):
    z32 = z.astype(jnp.int32)
    zr = jnp.pad(z32, (0, N_PAD - N_ATOMS)).reshape(GRID, 1, BLK)
    posT = jnp.pad(pos.T, ((0, 0), (0, N_PAD - N_ATOMS)))
    tableT = jnp.pad((emb_feat + b1[None, :]).T,
                     ((0, 0), (0, ZPAD - MAX_Z)))
    w1T = W1.T
    w2T = W2.T
    arefT = jnp.pad((atomref_table + b2).T, ((0, 0), (0, ZPAD - MAX_Z)))

    xT = pl.pallas_call(
        _tc_body,
        grid=(GRID,),
        in_specs=[
            pl.BlockSpec((1, 1, BLK), lambda i: (i, 0, 0)),
            pl.BlockSpec((3, BLK), lambda i: (0, i)),
            pl.BlockSpec((D_HID, ZPAD), lambda i: (0, 0)),
            pl.BlockSpec((D_HID, 3), lambda i: (0, 0)),
            pl.BlockSpec((1, D_HID), lambda i: (0, 0)),
            pl.BlockSpec((1, ZPAD), lambda i: (0, 0)),
        ],
        out_specs=pl.BlockSpec((1, 1, BLK), lambda i: (i, 0, 0)),
        out_shape=jax.ShapeDtypeStruct((GRID, 1, BLK), jnp.float32),
    )(zr, posT, tableT, w1T, w2T, arefT)

    x = xT.reshape(N_PAD)[:N_ATOMS].reshape(N_ATOMS, 1)
    return (x, z, pos, batch)
